# Initial kernel scaffold; baseline (speedup 1.0000x reference)
import jax
import jax.numpy as jnp
from jax.experimental import pallas as pl


def kernel(x, table):
    raise NotImplementedError("write your pallas kernel here")


# speedup vs baseline: 1.1424x; 1.1424x over previous
import functools

import jax
import jax.numpy as jnp
from jax import lax
from jax.experimental import pallas as pl
from jax.experimental.pallas import tpu as pltpu
from jax.experimental.pallas import tpu_sc as plsc

VOCAB_SIZE = 1000000
EMBED_SIZE = 32

_info = plsc.get_sparse_core_info()
_NC, _NS = _info.num_cores, _info.num_subcores
_NW = _NC * _NS

_B = 16384 * 50
_BPW = _B // _NW
_CH = 1024
_NCHUNK = _BPW // _CH


def _make_kernel():
  mesh = plsc.VectorSubcoreMesh(core_axis_name="c", subcore_axis_name="s")

  @functools.partial(
      pl.kernel,
      out_type=jax.ShapeDtypeStruct((_B, EMBED_SIZE), jnp.float32),
      mesh=mesh,
      scratch_types=[
          pltpu.VMEM((_CH,), jnp.int32),
          pltpu.VMEM((_CH, EMBED_SIZE), jnp.float32),
          pltpu.SemaphoreType.DMA,
      ],
      compiler_params=pltpu.CompilerParams(use_tc_tiling_on_sc=False),
  )
  def embed(idx_hbm, table_hbm, out_hbm, idx_v, rows_v, sem):
    wid = lax.axis_index("s") * _NC + lax.axis_index("c")
    base = wid * _BPW

    def body(c):
      off = base + c * _CH
      pltpu.sync_copy(idx_hbm.at[pl.ds(off, _CH)], idx_v)
      pltpu.async_copy(table_hbm.at[idx_v], rows_v, sem).wait()
      pltpu.sync_copy(rows_v, out_hbm.at[pl.ds(off, _CH)])

    pl.loop(0, _NCHUNK)(body)

  return embed


_embed = _make_kernel()


@jax.jit
def kernel(x, table):
  flat = x.reshape(-1).astype(jnp.int32)
  out = _embed(flat, table)
  return out.reshape(x.shape[0], x.shape[1], EMBED_SIZE)


# speedup vs baseline: 1.1628x; 1.0178x over previous
import functools

import jax
import jax.numpy as jnp
from jax import lax
from jax.experimental import pallas as pl
from jax.experimental.pallas import tpu as pltpu
from jax.experimental.pallas import tpu_sc as plsc

VOCAB_SIZE = 1000000
EMBED_SIZE = 32

_info = plsc.get_sparse_core_info()
_NC, _NS = _info.num_cores, _info.num_subcores
_NW = _NC * _NS

_B = 16384 * 50
_BPW = _B // _NW
_CH = 1280
_NCHUNK = _BPW // _CH


def _make_kernel():
  mesh = plsc.VectorSubcoreMesh(core_axis_name="c", subcore_axis_name="s")

  @functools.partial(
      pl.kernel,
      out_type=jax.ShapeDtypeStruct((_B, EMBED_SIZE), jnp.float32),
      mesh=mesh,
      scratch_types=[
          pltpu.VMEM((_NCHUNK, _CH), jnp.int32),
          pltpu.VMEM((_CH, EMBED_SIZE), jnp.float32),
          pltpu.VMEM((_CH, EMBED_SIZE), jnp.float32),
          pltpu.SemaphoreType.DMA,
          pltpu.SemaphoreType.DMA,
          pltpu.SemaphoreType.DMA,
          pltpu.SemaphoreType.DMA,
      ],
      compiler_params=pltpu.CompilerParams(use_tc_tiling_on_sc=False),
  )
  def embed(idx_hbm, table_hbm, out_hbm, idx_v, rows0, rows1, g0, g1, w0, w1):
    wid = lax.axis_index("s") * _NC + lax.axis_index("c")
    base = wid * _BPW
    rows = (rows0, rows1)
    gsem = (g0, g1)
    wsem = (w0, w1)

    pltpu.sync_copy(idx_hbm.at[wid], idx_v)

    gather_d = [None, None]
    write_d = [None, None]
    for c in range(_NCHUNK):
      s = c % 2
      if c >= 2:
        write_d[s].wait()
      gather_d[s] = pltpu.async_copy(table_hbm.at[idx_v.at[c]], rows[s], gsem[s])
      if c >= 1:
        p = 1 - s
        gather_d[p].wait()
        off = base + (c - 1) * _CH
        write_d[p] = pltpu.async_copy(rows[p], out_hbm.at[pl.ds(off, _CH)], wsem[p])
    s = (_NCHUNK - 1) % 2
    gather_d[s].wait()
    off = base + (_NCHUNK - 1) * _CH
    write_d[s] = pltpu.async_copy(rows[s], out_hbm.at[pl.ds(off, _CH)], wsem[s])
    write_d[0].wait()
    write_d[1].wait()

  return embed


_embed = _make_kernel()


@jax.jit
def kernel(x, table):
  flat = x.reshape(_NW, _NCHUNK, _CH).astype(jnp.int32)
  out = _embed(flat, table)
  return out.reshape(x.shape[0], x.shape[1], EMBED_SIZE)


# speedup vs baseline: 2.3232x; 1.9979x over previous
import functools

import jax
import jax.numpy as jnp
from jax import lax
from jax.experimental import pallas as pl
from jax.experimental.pallas import tpu as pltpu
from jax.experimental.pallas import tpu_sc as plsc

VOCAB_SIZE = 1000000
EMBED_SIZE = 32
SEQ = 50
BATCH = 16384

_info = plsc.get_sparse_core_info()
_NC, _NS = _info.num_cores, _info.num_subcores
_NW = _NC * _NS
_BW = BATCH // _NW
_TPW = _BW // 128


def _make_kernel():
  mesh = plsc.VectorSubcoreMesh(core_axis_name="c", subcore_axis_name="s")

  @functools.partial(
      pl.kernel,
      out_type=jax.ShapeDtypeStruct((SEQ, 4, BATCH // 128, 8, 128),
                                    jnp.float32),
      mesh=mesh,
      scratch_types=[
          pltpu.VMEM((SEQ, _BW), jnp.int32),
          pltpu.VMEM((_BW, EMBED_SIZE), jnp.float32),
          pltpu.VMEM((_BW, EMBED_SIZE), jnp.float32),
          pltpu.VMEM((4, _TPW, 8, 128), jnp.float32),
          pltpu.VMEM((4, _TPW, 8, 128), jnp.float32),
          pltpu.SemaphoreType.DMA,
          pltpu.SemaphoreType.DMA,
          pltpu.SemaphoreType.DMA,
          pltpu.SemaphoreType.DMA,
      ],
      compiler_params=pltpu.CompilerParams(use_tc_tiling_on_sc=False,
                                           needs_layout_passes=False),
  )
  def embed(xt_hbm, table_hbm, out_hbm, idx_v, rows0, rows1, ob0, ob1,
            g0, g1, w0, w1):
    wid = lax.axis_index("s") * _NC + lax.axis_index("c")
    b0 = wid * _BW
    btg0 = wid * _TPW
    rows = (rows0, rows1)
    obuf = (ob0, ob1)
    gsem = (g0, g1)
    wsem = (w0, w1)
    iota = lax.iota(jnp.int32, 16)

    pltpu.sync_copy(xt_hbm.at[:, pl.ds(b0, _BW)], idx_v)

    pltpu.async_copy(table_hbm.at[idx_v.at[0]], rows0, g0)
    pltpu.async_copy(table_hbm.at[idx_v.at[1]], rows1, g1)

    def outer(t):
      for b in range(2):
        s = 2 * t + b
        pltpu.make_async_copy(table_hbm.at[pl.ds(0, _BW)], rows[b],
                              gsem[b]).wait()

        @pl.when(t > 0)
        def _():
          pltpu.make_async_copy(obuf[b],
                                out_hbm.at[s, :, pl.ds(btg0, _TPW)],
                                wsem[b]).wait()

        @plsc.parallel_loop(0, 4 * _TPW * 8, unroll=2)
        def transpose(j):
          dt = j >> 5
          bt = (j >> 3) & (_TPW - 1)
          dr = j & 7
          col = jnp.full((16,), dt * 8 + dr, jnp.int32)
          for g in range(8):
            rowv = bt * 128 + g * 16 + iota
            vals = plsc.load_gather(rows[b], [rowv, col])
            obuf[b][dt, bt, dr, pl.ds(g * 16, 16)] = vals

        pltpu.async_copy(obuf[b], out_hbm.at[s, :, pl.ds(btg0, _TPW)],
                         wsem[b])

        @pl.when(s + 2 < SEQ)
        def _():
          pltpu.async_copy(table_hbm.at[idx_v.at[s + 2]], rows[b], gsem[b])

    pl.loop(0, SEQ // 2)(outer)

    for b in range(2):
      pltpu.make_async_copy(obuf[b], out_hbm.at[SEQ - 2 + b, :,
                                                pl.ds(btg0, _TPW)],
                            wsem[b]).wait()

  return embed


_embed = _make_kernel()


@jax.jit
def kernel(x, table):
  xt = jnp.transpose(x).astype(jnp.int32)
  out5d = _embed(xt, table)
  return out5d.transpose(2, 4, 0, 1, 3).reshape(BATCH, SEQ, EMBED_SIZE)
